# baseline (speedup 1.0000x reference)
import functools

import jax
import jax.numpy as jnp
from jax import lax
from jax.experimental import pallas as pl
from jax.experimental.pallas import tpu as pltpu

_N = 10000
_E = 320000
_NP = 10240
_C = 2048
_G = 162
_EP = _C * _G
_RB = 320


def _k1_body(x_ref, w_ref, asrc_ref, adst_ref, h_ref, as_ref, ad_ref):
    h = jnp.dot(x_ref[...], w_ref[...], preferred_element_type=jnp.float32)
    h_ref[...] = h
    as_ref[...] = jnp.dot(h, asrc_ref[...], preferred_element_type=jnp.float32)
    ad_ref[...] = jnp.dot(h, adst_ref[...], preferred_element_type=jnp.float32)


def _dense_stage(xp, W, A_src8, A_dst8):
    d_in = xp.shape[1]
    d_out = W.shape[1]
    grid = _NP // _RB
    return pl.pallas_call(
        _k1_body,
        grid=(grid,),
        in_specs=[
            pl.BlockSpec((_RB, d_in), lambda g: (g, 0)),
            pl.BlockSpec((d_in, d_out), lambda g: (0, 0)),
            pl.BlockSpec((d_out, 8), lambda g: (0, 0)),
            pl.BlockSpec((d_out, 8), lambda g: (0, 0)),
        ],
        out_specs=[
            pl.BlockSpec((_RB, d_out), lambda g: (g, 0)),
            pl.BlockSpec((_RB, 8), lambda g: (g, 0)),
            pl.BlockSpec((_RB, 8), lambda g: (g, 0)),
        ],
        out_shape=[
            jax.ShapeDtypeStruct((_NP, d_out), jnp.float32),
            jax.ShapeDtypeStruct((_NP, 8), jnp.float32),
            jax.ShapeDtypeStruct((_NP, 8), jnp.float32),
        ],
    )(xp, W, A_src8, A_dst8)


def _k2_body(src_ref, dst_ref, as_ref, ad_ref, ex_ref, den_ref):
    g = pl.program_id(0)

    @pl.when(g == 0)
    def _():
        den_ref[...] = jnp.zeros_like(den_ref)

    def body(i, _):
        s = src_ref[0, 0, i]
        d = dst_ref[0, 0, i]
        e = as_ref[pl.ds(s, 1), :] + ad_ref[pl.ds(d, 1), :]
        e = jnp.where(e > 0, e, 0.2 * e)
        x = jnp.exp(e)
        ex_ref[pl.ds(i, 1), :] = x
        den_ref[pl.ds(d, 1), :] = den_ref[pl.ds(d, 1), :] + x
        return 0

    lax.fori_loop(0, _C, body, 0)


def _edge_softmax_stage(src3, dst3, as8, ad8):
    return pl.pallas_call(
        _k2_body,
        grid=(_G,),
        in_specs=[
            pl.BlockSpec((1, 1, _C), lambda g: (g, 0, 0), memory_space=pltpu.SMEM),
            pl.BlockSpec((1, 1, _C), lambda g: (g, 0, 0), memory_space=pltpu.SMEM),
            pl.BlockSpec((_NP, 8), lambda g: (0, 0)),
            pl.BlockSpec((_NP, 8), lambda g: (0, 0)),
        ],
        out_specs=[
            pl.BlockSpec((_C, 8), lambda g: (g, 0)),
            pl.BlockSpec((_NP, 8), lambda g: (0, 0)),
        ],
        out_shape=[
            jax.ShapeDtypeStruct((_EP, 8), jnp.float32),
            jax.ShapeDtypeStruct((_NP, 8), jnp.float32),
        ],
    )(src3, dst3, as8, ad8)


def _sel_by_p(p, vals):
    out = jnp.float32(vals[-1]) if isinstance(vals[-1], float) else vals[-1]
    for q in range(len(vals) - 2, -1, -1):
        out = jnp.where(p == q, vals[q], out)
    return out


def _k3_body(src_ref, dst_ref, ex_ref, den_ref, h_ref, b_ref, out_ref, *,
             fpart, thr, head_lo, head_hi):
    p = pl.program_id(0)
    g = pl.program_id(1)

    @pl.when(g == 0)
    def _():
        out_ref[...] = jnp.zeros_like(out_ref)

    lane = lax.broadcasted_iota(jnp.int32, (1, fpart), 1)

    def body(i, _):
        s = src_ref[0, 0, i]
        d = dst_ref[0, 0, i]
        ex = ex_ref[pl.ds(i, 1), :]
        den = den_ref[pl.ds(d, 1), :]
        al = ex / (den + 1e-16)
        a_lo = _sel_by_p(p, [al[0, k] for k in head_lo])
        a_hi = _sel_by_p(p, [al[0, k] for k in head_hi])
        t = _sel_by_p(p, [jnp.int32(v) for v in thr])
        scale = jnp.where(lane < t, a_lo, a_hi)
        msg = h_ref[pl.ds(s, 1), :] * scale
        out_ref[pl.ds(d, 1), :] = out_ref[pl.ds(d, 1), :] + msg
        return 0

    lax.fori_loop(0, _C, body, 0)

    @pl.when(g == _G - 1)
    def _():
        out_ref[...] = jnp.maximum(out_ref[...] + b_ref[pl.ds(0, 1), :], 0.0)


def _message_stage(src3, dst3, ex, den, h, b8, thr, head_lo, head_hi):
    d_out = h.shape[1]
    nparts = len(thr)
    fpart = d_out // nparts
    body = functools.partial(_k3_body, fpart=fpart, thr=thr,
                             head_lo=head_lo, head_hi=head_hi)
    return pl.pallas_call(
        body,
        grid=(nparts, _G),
        in_specs=[
            pl.BlockSpec((1, 1, _C), lambda p, g: (g, 0, 0), memory_space=pltpu.SMEM),
            pl.BlockSpec((1, 1, _C), lambda p, g: (g, 0, 0), memory_space=pltpu.SMEM),
            pl.BlockSpec((_C, 8), lambda p, g: (g, 0)),
            pl.BlockSpec((_NP, 8), lambda p, g: (0, 0)),
            pl.BlockSpec((_NP, fpart), lambda p, g: (0, p)),
            pl.BlockSpec((8, fpart), lambda p, g: (0, p)),
        ],
        out_specs=pl.BlockSpec((_NP, fpart), lambda p, g: (0, p)),
        out_shape=jax.ShapeDtypeStruct((_NP, d_out), jnp.float32),
    )(src3, dst3, ex, den, h, b8)


def _pad_logits(a):
    heads, hid = a.shape
    cols = [a.reshape(-1) * (jnp.arange(heads)[:, None] == k).astype(a.dtype)
            .repeat(hid).reshape(-1) for k in range(heads)]
    m = jnp.stack(cols, axis=1)
    return jnp.pad(m, ((0, 0), (0, 8 - heads)))


def _gat_layer(xp, src3, dst3, W, a_src, a_dst, b, thr, head_lo, head_hi):
    A_src8 = _pad_logits(a_src)
    A_dst8 = _pad_logits(a_dst)
    h, as8, ad8 = _dense_stage(xp, W, A_src8, A_dst8)
    ex, den = _edge_softmax_stage(src3, dst3, as8, ad8)
    b8 = jnp.broadcast_to(b[None, :], (8, b.shape[0]))
    return _message_stage(src3, dst3, ex, den, h, b8, thr, head_lo, head_hi)


def kernel(x, edge_index, W1, a_src1, a_dst1, b1, W2, a_src2, a_dst2, b2):
    loops = jnp.arange(_N, dtype=edge_index.dtype)
    pad = jnp.full((_EP - _E - _N,), _N, dtype=edge_index.dtype)
    src3 = jnp.concatenate([edge_index[0], loops, pad]).reshape(_G, 1, _C)
    dst3 = jnp.concatenate([edge_index[1], loops, pad]).reshape(_G, 1, _C)
    xp = jnp.pad(x, ((0, _NP - _N), (0, 0)))
    h1 = _gat_layer(xp, src3, dst3, W1, a_src1, a_dst1, b1,
                    thr=[256, 256, 256],
                    head_lo=[0, 1, 2], head_hi=[0, 1, 2])
    h2 = _gat_layer(h1, src3, dst3, W2, a_src2, a_dst2, b2,
                    thr=[128, 128, 128], head_lo=[0, 0, 0], head_hi=[0, 0, 0])
    return h2[:_N]
